# Initial kernel scaffold; baseline (speedup 1.0000x reference)
import jax
import jax.numpy as jnp
from jax.experimental import pallas as pl


def kernel(s, z, aw_W1, aw_b1, aw_W2, aw_b2, aw_W3, aw_b3, av_W1, av_b1, av_W2, av_b2, av_W3, av_b3, ao_W, ao_b, bn1_g, bn1_b, bn1_m, bn1_v, ff_W1, ff_b1, ff_W2, ff_b2, bn2_g, bn2_b, bn2_m, bn2_v, edge_idx):
    raise NotImplementedError("write your pallas kernel here")


# speedup vs baseline: 7.2250x; 7.2250x over previous
import functools

import jax
import jax.numpy as jnp
from jax.experimental import pallas as pl


def _gelu(x):
    return x * 0.5 * (1.0 + jax.lax.erf(x * 0.7071067811865476))


def _nodeproj_body(s_ref, w_ref, o_ref):
    o_ref[...] = jnp.dot(s_ref[...], w_ref[...],
                         preferred_element_type=jnp.float32)


def _node_proj(s, w, tile=1000):
    n, d = s.shape
    hd = w.shape[1]
    grid = n // tile
    return pl.pallas_call(
        _nodeproj_body,
        grid=(grid,),
        in_specs=[
            pl.BlockSpec((tile, d), lambda i: (i, 0)),
            pl.BlockSpec((d, hd), lambda i: (0, 0)),
        ],
        out_specs=pl.BlockSpec((tile, hd), lambda i: (i, 0)),
        out_shape=jax.ShapeDtypeStruct((n, hd), jnp.float32),
    )(s, w)


def _edge_body(sg_ref, z_ref, w1z_ref, b1_ref, w2_ref, b2_ref, w3_ref, b3_ref,
               vw1_ref, vb1_ref, vw2_ref, vb2_ref, vw3_ref, vb3_ref,
               aw_ref, v_ref, gmax_ref):
    z = z_ref[...]
    h = _gelu(sg_ref[...] + jnp.dot(z, w1z_ref[...],
                                    preferred_element_type=jnp.float32)
              + b1_ref[...])
    h = _gelu(jnp.dot(h, w2_ref[...], preferred_element_type=jnp.float32)
              + b2_ref[...])
    aw = jnp.dot(h, w3_ref[...], preferred_element_type=jnp.float32) + b3_ref[...]
    aw_ref[...] = aw

    v = _gelu(jnp.dot(z, vw1_ref[...], preferred_element_type=jnp.float32)
              + vb1_ref[...])
    v = _gelu(jnp.dot(v, vw2_ref[...], preferred_element_type=jnp.float32)
              + vb2_ref[...])
    v_ref[...] = (jnp.dot(v, vw3_ref[...], preferred_element_type=jnp.float32)
                  + vb3_ref[...])

    tile_max = jnp.max(aw, axis=0, keepdims=True)

    @pl.when(pl.program_id(0) == 0)
    def _init():
        gmax_ref[...] = jnp.full_like(gmax_ref, -jnp.inf)

    gmax_ref[...] = jnp.maximum(gmax_ref[...], tile_max)


def _edge_mlp(sg, z, w1z, b1, w2, b2, w3, b3, vw1, vb1, vw2, vb2, vw3, vb3,
              tile=640):
    e, d = sg.shape
    dp = z.shape[1]
    heads = w3.shape[1]
    grid = e // tile
    full = lambda i: (0, 0)
    return pl.pallas_call(
        _edge_body,
        grid=(grid,),
        in_specs=[
            pl.BlockSpec((tile, d), lambda i: (i, 0)),
            pl.BlockSpec((tile, dp), lambda i: (i, 0)),
            pl.BlockSpec(w1z.shape, full), pl.BlockSpec(b1.shape, full),
            pl.BlockSpec(w2.shape, full), pl.BlockSpec(b2.shape, full),
            pl.BlockSpec(w3.shape, full), pl.BlockSpec(b3.shape, full),
            pl.BlockSpec(vw1.shape, full), pl.BlockSpec(vb1.shape, full),
            pl.BlockSpec(vw2.shape, full), pl.BlockSpec(vb2.shape, full),
            pl.BlockSpec(vw3.shape, full), pl.BlockSpec(vb3.shape, full),
        ],
        out_specs=[
            pl.BlockSpec((tile, heads), lambda i: (i, 0)),
            pl.BlockSpec((tile, d), lambda i: (i, 0)),
            pl.BlockSpec((1, heads), full),
        ],
        out_shape=[
            jax.ShapeDtypeStruct((e, heads), jnp.float32),
            jax.ShapeDtypeStruct((e, d), jnp.float32),
            jax.ShapeDtypeStruct((1, heads), jnp.float32),
        ],
    )(sg, z, w1z, b1, w2, b2, w3, b3, vw1, vb1, vw2, vb2, vw3, vb3)


def _wval_body(attn_ref, v_ref, wo_ref, p_ref):
    v = v_ref[...]
    attn = attn_ref[...]
    heads = attn.shape[1]
    acc = jnp.zeros_like(v)
    for h in range(heads):
        vp = jnp.dot(v, wo_ref[h], preferred_element_type=jnp.float32)
        acc = acc + attn[:, h:h + 1] * vp
    p_ref[...] = acc


def _weighted_proj(attn, v, wo, tile=640):
    e, d = v.shape
    heads = attn.shape[1]
    grid = e // tile
    return pl.pallas_call(
        _wval_body,
        grid=(grid,),
        in_specs=[
            pl.BlockSpec((tile, heads), lambda i: (i, 0)),
            pl.BlockSpec((tile, d), lambda i: (i, 0)),
            pl.BlockSpec(wo.shape, lambda i: (0, 0, 0)),
        ],
        out_specs=pl.BlockSpec((tile, d), lambda i: (i, 0)),
        out_shape=jax.ShapeDtypeStruct((e, d), jnp.float32),
    )(attn, v, wo)


def _final_body(oagg_ref, s_ref, ob_ref, k1_ref, c1_ref,
                fw1_ref, fb1_ref, fw2_ref, fb2_ref, k2_ref, c2_ref, out_ref):
    o = oagg_ref[...] + ob_ref[...]
    s1 = s_ref[...] + o * k1_ref[...] + c1_ref[...]
    f = _gelu(jnp.dot(s1, fw1_ref[...], preferred_element_type=jnp.float32)
              + fb1_ref[...])
    f = jnp.dot(f, fw2_ref[...], preferred_element_type=jnp.float32) + fb2_ref[...]
    out_ref[...] = s1 + f * k2_ref[...] + c2_ref[...]


def _final_stage(oagg, s, ob, k1, c1, fw1, fb1, fw2, fb2, k2, c2, tile=1000):
    n, d = s.shape
    grid = n // tile
    full = lambda i: (0, 0)
    return pl.pallas_call(
        _final_body,
        grid=(grid,),
        in_specs=[
            pl.BlockSpec((tile, d), lambda i: (i, 0)),
            pl.BlockSpec((tile, d), lambda i: (i, 0)),
            pl.BlockSpec(ob.shape, full), pl.BlockSpec(k1.shape, full),
            pl.BlockSpec(c1.shape, full), pl.BlockSpec(fw1.shape, full),
            pl.BlockSpec(fb1.shape, full), pl.BlockSpec(fw2.shape, full),
            pl.BlockSpec(fb2.shape, full), pl.BlockSpec(k2.shape, full),
            pl.BlockSpec(c2.shape, full),
        ],
        out_specs=pl.BlockSpec((tile, d), lambda i: (i, 0)),
        out_shape=jax.ShapeDtypeStruct((n, d), jnp.float32),
    )(oagg, s, ob, k1, c1, fw1, fb1, fw2, fb2, k2, c2)


def kernel(s, z, aw_W1, aw_b1, aw_W2, aw_b2, aw_W3, aw_b3,
           av_W1, av_b1, av_W2, av_b2, av_W3, av_b3,
           ao_W, ao_b, bn1_g, bn1_b, bn1_m, bn1_v,
           ff_W1, ff_b1, ff_W2, ff_b2, bn2_g, bn2_b, bn2_m, bn2_v,
           edge_idx):
    n, d = s.shape
    e = z.shape[0]
    heads = aw_W3.shape[1]
    dst = edge_idx[1]

    row = lambda x: x.reshape(1, -1)

    su = _node_proj(s, aw_W1[:d])

    sg = jnp.take(su, dst, axis=0)

    aw, v, gmax = _edge_mlp(
        sg, z, aw_W1[d:], row(aw_b1), aw_W2, row(aw_b2), aw_W3, row(aw_b3),
        av_W1, row(av_b1), av_W2, row(av_b2), av_W3, row(av_b3))

    w = jnp.exp(aw - gmax)
    denom = jax.ops.segment_sum(w, dst, num_segments=n)
    attn = w * (1.0 / denom)[dst]

    wo = ao_W.reshape(heads, d, d)
    p = _weighted_proj(attn, v, wo)

    oagg = jax.ops.segment_sum(p, dst, num_segments=n)

    k1 = bn1_g / jnp.sqrt(bn1_v + 1e-5)
    c1 = bn1_b - bn1_m * k1
    k2 = bn2_g / jnp.sqrt(bn2_v + 1e-5)
    c2 = bn2_b - bn2_m * k2
    return _final_stage(oagg, s, row(ao_b), row(k1), row(c1),
                        ff_W1, row(ff_b1), ff_W2, row(ff_b2), row(k2), row(c2))
